# baseline (speedup 1.0000x reference)
import functools

import jax
import jax.numpy as jnp
from jax import lax
from jax.experimental import pallas as pl
from jax.experimental.pallas import tpu as pltpu
from jax.experimental.pallas import tpu_sc as plsc

N = 10000
E = 320000
NPAD = 10240
NACC = 10112
B = 128
NBH = 80
NTILES = 32
ETOT = NTILES * NBH * B
EPAD = ETOT - E
NDEAD = 112
BR = 1000
F_IN = 128
H = 256
C = 40
CP = 128

_MESH = plsc.VectorSubcoreMesh(core_axis_name="c", subcore_axis_name="s")
_SC_PARAMS = pltpu.CompilerParams(needs_layout_passes=False)


def _sc_hist(dst3, zrows):
    @functools.partial(
        pl.kernel,
        out_type=jax.ShapeDtypeStruct((NTILES, 640, 16), jnp.float32),
        mesh=_MESH,
        compiler_params=_SC_PARAMS,
        scratch_types=[
            pltpu.VMEM((NBH, B), jnp.int32),
            pltpu.VMEM((640, 16), jnp.float32),
        ],
    )
    def k(dst_hbm, z_hbm, out_hbm, dbuf, hist):
        c = lax.axis_index("c")
        s = lax.axis_index("s")
        t = c * 16 + s
        pltpu.sync_copy(dst_hbm.at[t], dbuf)
        pltpu.sync_copy(z_hbm, hist)

        @pl.loop(0, NBH)
        def _(j):
            @pl.loop(0, B, step=16)
            def _(kk):
                d = dbuf[j, pl.ds(kk, 16)]
                row = lax.shift_right_logical(d, 4)
                lane = lax.bitwise_and(d, 15)
                plsc.addupdate_scatter(
                    hist, [row, lane], jnp.ones((16,), jnp.float32))

        pltpu.sync_copy(hist, out_hbm.at[t])

    return k(dst3, zrows)


def _sc_prop(table, src3, dst3, zrows, width):
    @functools.partial(
        pl.kernel,
        out_type=jax.ShapeDtypeStruct((2, NACC, width), jnp.float32),
        mesh=_MESH,
        scratch_types=[
            pltpu.VMEM((NBH, B), jnp.int32),
            pltpu.VMEM((NBH, B), jnp.int32),
            pltpu.VMEM((B, width), jnp.float32),
            pltpu.VMEM_SHARED((NACC, width), jnp.float32),
            pltpu.SemaphoreType.DMA,
        ],
    )
    def k(tab_hbm, src_hbm, dst_hbm, z_hbm, out_hbm,
          sbuf, dbuf, rows, accum, gsem):
        c = lax.axis_index("c")
        s = lax.axis_index("s")
        t = c * 16 + s
        pltpu.sync_copy(src_hbm.at[t], sbuf)
        pltpu.sync_copy(dst_hbm.at[t], dbuf)
        pltpu.sync_copy(z_hbm, accum.at[pl.ds(s * 632, 632)])
        plsc.subcore_barrier()

        @pl.loop(0, NBH)
        def _(j):
            pltpu.async_copy(tab_hbm.at[sbuf.at[j]], rows, gsem).wait()
            pltpu.sync_copy(rows, accum.at[dbuf.at[j]], add=True)

        plsc.subcore_barrier()
        pltpu.sync_copy(accum.at[pl.ds(s * 632, 632)],
                        out_hbm.at[c, pl.ds(s * 632, 632)])

    return k(table, src3, dst3, zrows)


def _tc_prep(hist, x):
    def body(h_ref, x_ref, dinv_ref, xt_ref):
        deg = jnp.sum(h_ref[...], axis=0) + 1.0
        dinv = lax.rsqrt(deg)
        dinv_ref[...] = dinv
        xt_ref[...] = x_ref[...] * dinv

    return pl.pallas_call(
        body,
        grid=(N // BR,),
        in_specs=[
            pl.BlockSpec((NTILES, BR, 1), lambda i: (0, i, 0)),
            pl.BlockSpec((BR, F_IN), lambda i: (i, 0)),
        ],
        out_specs=[
            pl.BlockSpec((BR, 1), lambda i: (i, 0)),
            pl.BlockSpec((BR, F_IN), lambda i: (i, 0)),
        ],
        out_shape=[
            jax.ShapeDtypeStruct((N, 1), jnp.float32),
            jax.ShapeDtypeStruct((N, F_IN), jnp.float32),
        ],
    )(hist, x)


def _bn_affine(b, g, be, rm, rv):
    sc = g * lax.rsqrt(rv + 1e-5)
    return sc, (b - rm) * sc + be


def _tc_layer1(parts, xt, dinv, W1, b1, g1, be1, rm1, rv1):
    def body(p_ref, xt_ref, dinv_ref, w_ref, b_ref, g_ref, be_ref, rm_ref,
             rv_ref, ha_ref, hb_ref):
        dinv = dinv_ref[...]
        agg = (p_ref[0] + p_ref[1] + xt_ref[...]) * dinv
        z = jnp.dot(agg, w_ref[...], preferred_element_type=jnp.float32)
        sc, sh = _bn_affine(b_ref[...], g_ref[...], be_ref[...], rm_ref[...],
                            rv_ref[...])
        h = jnp.maximum(z * sc + sh, 0.0) * dinv
        ha_ref[...] = h[:, :F_IN]
        hb_ref[...] = h[:, F_IN:]

    p_spec = pl.BlockSpec((2, BR, F_IN), lambda i: (0, i, 0))
    v_spec = pl.BlockSpec((1, H), lambda i: (0, 0))
    return pl.pallas_call(
        body,
        grid=(N // BR,),
        in_specs=[
            p_spec,
            pl.BlockSpec((BR, F_IN), lambda i: (i, 0)),
            pl.BlockSpec((BR, 1), lambda i: (i, 0)),
            pl.BlockSpec((F_IN, H), lambda i: (0, 0)),
            v_spec, v_spec, v_spec, v_spec, v_spec,
        ],
        out_specs=[
            pl.BlockSpec((BR, F_IN), lambda i: (i, 0)),
            pl.BlockSpec((BR, F_IN), lambda i: (i, 0)),
        ],
        out_shape=[
            jax.ShapeDtypeStruct((N, F_IN), jnp.float32),
            jax.ShapeDtypeStruct((N, F_IN), jnp.float32),
        ],
    )(parts, xt, dinv, W1, b1, g1, be1, rm1, rv1)


def _tc_layer2(qa, qb, ha, hb, dinv, W2, b2, g2, be2, rm2, rv2, W3p):
    def body(qa_ref, qb_ref, ha_ref, hb_ref, dinv_ref, w2_ref, b_ref, g_ref,
             be_ref, rm_ref, rv_ref, w3_ref, tt_ref):
        dinv = dinv_ref[...]
        agg_a = (qa_ref[0] + qa_ref[1] + ha_ref[...]) * dinv
        agg_b = (qb_ref[0] + qb_ref[1] + hb_ref[...]) * dinv
        agg = jnp.concatenate([agg_a, agg_b], axis=1)
        z = jnp.dot(agg, w2_ref[...], preferred_element_type=jnp.float32)
        sc, sh = _bn_affine(b_ref[...], g_ref[...], be_ref[...], rm_ref[...],
                            rv_ref[...])
        h2 = jnp.maximum(z * sc + sh, 0.0)
        t = jnp.dot(h2, w3_ref[...], preferred_element_type=jnp.float32)
        tt_ref[...] = t * dinv

    p_spec = pl.BlockSpec((2, BR, F_IN), lambda i: (0, i, 0))
    h_spec = pl.BlockSpec((BR, F_IN), lambda i: (i, 0))
    v_spec = pl.BlockSpec((1, H), lambda i: (0, 0))
    return pl.pallas_call(
        body,
        grid=(N // BR,),
        in_specs=[
            p_spec, p_spec, h_spec, h_spec,
            pl.BlockSpec((BR, 1), lambda i: (i, 0)),
            pl.BlockSpec((H, H), lambda i: (0, 0)),
            v_spec, v_spec, v_spec, v_spec, v_spec,
            pl.BlockSpec((H, CP), lambda i: (0, 0)),
        ],
        out_specs=pl.BlockSpec((BR, CP), lambda i: (i, 0)),
        out_shape=jax.ShapeDtypeStruct((N, CP), jnp.float32),
    )(qa, qb, ha, hb, dinv, W2, b2, g2, be2, rm2, rv2, W3p)


def _tc_layer3(r, tt, dinv, b3p):
    def body(r_ref, tt_ref, dinv_ref, b_ref, out_ref):
        agg = (r_ref[0] + r_ref[1] + tt_ref[...]) * dinv_ref[...]
        logits = agg + b_ref[...]
        col = lax.broadcasted_iota(jnp.int32, (BR, CP), 1)
        masked = jnp.where(col < C, logits, -1e30)
        m = jnp.max(masked, axis=1, keepdims=True)
        lse = jnp.log(jnp.sum(jnp.exp(masked - m), axis=1, keepdims=True))
        out_ref[...] = logits - m - lse

    return pl.pallas_call(
        body,
        grid=(N // BR,),
        in_specs=[
            pl.BlockSpec((2, BR, CP), lambda i: (0, i, 0)),
            pl.BlockSpec((BR, CP), lambda i: (i, 0)),
            pl.BlockSpec((BR, 1), lambda i: (i, 0)),
            pl.BlockSpec((1, CP), lambda i: (0, 0)),
        ],
        out_specs=pl.BlockSpec((BR, CP), lambda i: (i, 0)),
        out_shape=jax.ShapeDtypeStruct((N, CP), jnp.float32),
    )(r, tt, dinv, b3p)


def kernel(x, W1, b1, g1, be1, rm1, rv1, W2, b2, g2, be2, rm2, rv2, W3, b3,
           edge_index):
    ei = edge_index.astype(jnp.int32)
    dead = N + jnp.arange(EPAD, dtype=jnp.int32) % NDEAD
    src = jnp.concatenate([ei[0], jnp.zeros((EPAD,), jnp.int32)])
    dst = jnp.concatenate([ei[1], dead])
    dst3h = dst.reshape(NTILES, NBH, B)
    src3 = src.reshape(NTILES, NBH, B)
    dst3 = dst3h

    z16 = jnp.zeros((640, 16), jnp.float32)
    z128 = jnp.zeros((632, F_IN), jnp.float32)

    hist = _sc_hist(dst3h, z16).reshape(NTILES, NPAD, 1)
    dinv, xt = _tc_prep(hist, x)

    p = _sc_prop(xt, src3, dst3, z128, F_IN)
    ha, hb = _tc_layer1(p, xt, dinv, W1,
                        b1.reshape(1, H), g1.reshape(1, H),
                        be1.reshape(1, H), rm1.reshape(1, H),
                        rv1.reshape(1, H))

    qa = _sc_prop(ha, src3, dst3, z128, F_IN)
    qb = _sc_prop(hb, src3, dst3, z128, F_IN)
    W3p = jnp.pad(W3, ((0, 0), (0, CP - C)))
    tt = _tc_layer2(qa, qb, ha, hb, dinv, W2,
                    b2.reshape(1, H), g2.reshape(1, H), be2.reshape(1, H),
                    rm2.reshape(1, H), rv2.reshape(1, H), W3p)

    r = _sc_prop(tt, src3, dst3, z128, CP)
    b3p = jnp.pad(b3, (0, CP - C)).reshape(1, CP)
    out = _tc_layer3(r, tt, dinv, b3p)
    return out[:, :C]


# speedup vs baseline: 2.4087x; 2.4087x over previous
import functools

import jax
import jax.numpy as jnp
from jax import lax
from jax.experimental import pallas as pl
from jax.experimental.pallas import tpu as pltpu
from jax.experimental.pallas import tpu_sc as plsc

N = 10000
E = 320000
NPAD = 10240
NACC = 10112
B = 128
NBH = 80
NTILES = 32
ETOT = NTILES * NBH * B
EPAD = ETOT - E
NDEAD = 112
BR = 1000
F_IN = 128
H = 256
C = 40
CP = 128

_MESH = plsc.VectorSubcoreMesh(core_axis_name="c", subcore_axis_name="s")
_SC_PARAMS = pltpu.CompilerParams(needs_layout_passes=False)


def _sc_hist(dst3, zrows):
    @functools.partial(
        pl.kernel,
        out_type=jax.ShapeDtypeStruct((NTILES, 640, 16), jnp.float32),
        mesh=_MESH,
        compiler_params=_SC_PARAMS,
        scratch_types=[
            pltpu.VMEM((NBH, B), jnp.int32),
            pltpu.VMEM((640, 16), jnp.float32),
        ],
    )
    def k(dst_hbm, z_hbm, out_hbm, dbuf, hist):
        c = lax.axis_index("c")
        s = lax.axis_index("s")
        t = c * 16 + s
        pltpu.sync_copy(dst_hbm.at[t], dbuf)
        pltpu.sync_copy(z_hbm, hist)

        @pl.loop(0, NBH)
        def _(j):
            @pl.loop(0, B, step=16)
            def _(kk):
                d = dbuf[j, pl.ds(kk, 16)]
                row = lax.shift_right_logical(d, 4)
                lane = lax.bitwise_and(d, 15)
                plsc.addupdate_scatter(
                    hist, [row, lane], jnp.ones((16,), jnp.float32))

        pltpu.sync_copy(hist, out_hbm.at[t])

    return k(dst3, zrows)


def _sc_prop(table, src3, dst3, zrows, width):
    @functools.partial(
        pl.kernel,
        out_type=jax.ShapeDtypeStruct((2, NACC, width), jnp.float32),
        mesh=_MESH,
        scratch_types=[
            pltpu.VMEM((NBH, B), jnp.int32),
            pltpu.VMEM((NBH, B), jnp.int32),
            pltpu.VMEM((B, width), jnp.float32),
            pltpu.VMEM_SHARED((NACC, width), jnp.float32),
            pltpu.SemaphoreType.DMA,
        ],
    )
    def k(tab_hbm, src_hbm, dst_hbm, z_hbm, out_hbm,
          sbuf, dbuf, rows, accum, gsem):
        c = lax.axis_index("c")
        s = lax.axis_index("s")
        t = c * 16 + s
        pltpu.sync_copy(src_hbm.at[t], sbuf)
        pltpu.sync_copy(dst_hbm.at[t], dbuf)
        pltpu.sync_copy(z_hbm, accum.at[pl.ds(s * 632, 632)])
        plsc.subcore_barrier()

        @pl.loop(0, NBH)
        def _(j):
            pltpu.async_copy(tab_hbm.at[sbuf.at[j]], rows, gsem).wait()
            pltpu.sync_copy(rows, accum.at[dbuf.at[j]], add=True)

        plsc.subcore_barrier()
        pltpu.sync_copy(accum.at[pl.ds(s * 632, 632)],
                        out_hbm.at[c, pl.ds(s * 632, 632)])

    return k(table, src3, dst3, zrows)


def _tc_prep(hist, x):
    def body(h_ref, x_ref, dinv_ref, xt_ref):
        deg = jnp.sum(h_ref[...], axis=0) + 1.0
        dinv = lax.rsqrt(deg)
        dinv_ref[...] = dinv
        xt_ref[...] = x_ref[...] * dinv

    return pl.pallas_call(
        body,
        grid=(N // BR,),
        in_specs=[
            pl.BlockSpec((NTILES, BR, 1), lambda i: (0, i, 0)),
            pl.BlockSpec((BR, F_IN), lambda i: (i, 0)),
        ],
        out_specs=[
            pl.BlockSpec((BR, 1), lambda i: (i, 0)),
            pl.BlockSpec((BR, F_IN), lambda i: (i, 0)),
        ],
        out_shape=[
            jax.ShapeDtypeStruct((N, 1), jnp.float32),
            jax.ShapeDtypeStruct((N, F_IN), jnp.float32),
        ],
    )(hist, x)


def _bn_affine(b, g, be, rm, rv):
    sc = g * lax.rsqrt(rv + 1e-5)
    return sc, (b - rm) * sc + be


def _tc_layer1(parts, xt, dinv, W1, b1, g1, be1, rm1, rv1):
    def body(p_ref, xt_ref, dinv_ref, w_ref, b_ref, g_ref, be_ref, rm_ref,
             rv_ref, ha_ref, hb_ref):
        dinv = dinv_ref[...]
        agg = (p_ref[0] + p_ref[1] + xt_ref[...]) * dinv
        z = jnp.dot(agg, w_ref[...], preferred_element_type=jnp.float32)
        sc, sh = _bn_affine(b_ref[...], g_ref[...], be_ref[...], rm_ref[...],
                            rv_ref[...])
        h = jnp.maximum(z * sc + sh, 0.0) * dinv
        ha_ref[...] = h[:, :F_IN]
        hb_ref[...] = h[:, F_IN:]

    p_spec = pl.BlockSpec((2, BR, F_IN), lambda i: (0, i, 0))
    v_spec = pl.BlockSpec((1, H), lambda i: (0, 0))
    return pl.pallas_call(
        body,
        grid=(N // BR,),
        in_specs=[
            p_spec,
            pl.BlockSpec((BR, F_IN), lambda i: (i, 0)),
            pl.BlockSpec((BR, 1), lambda i: (i, 0)),
            pl.BlockSpec((F_IN, H), lambda i: (0, 0)),
            v_spec, v_spec, v_spec, v_spec, v_spec,
        ],
        out_specs=[
            pl.BlockSpec((BR, F_IN), lambda i: (i, 0)),
            pl.BlockSpec((BR, F_IN), lambda i: (i, 0)),
        ],
        out_shape=[
            jax.ShapeDtypeStruct((N, F_IN), jnp.float32),
            jax.ShapeDtypeStruct((N, F_IN), jnp.float32),
        ],
    )(parts, xt, dinv, W1, b1, g1, be1, rm1, rv1)


def _tc_layer2(qa, qb, ha, hb, dinv, W2, b2, g2, be2, rm2, rv2, W3p):
    def body(qa_ref, qb_ref, ha_ref, hb_ref, dinv_ref, w2_ref, b_ref, g_ref,
             be_ref, rm_ref, rv_ref, w3_ref, tt_ref):
        dinv = dinv_ref[...]
        agg_a = (qa_ref[0] + qa_ref[1] + ha_ref[...]) * dinv
        agg_b = (qb_ref[0] + qb_ref[1] + hb_ref[...]) * dinv
        agg = jnp.concatenate([agg_a, agg_b], axis=1)
        z = jnp.dot(agg, w2_ref[...], preferred_element_type=jnp.float32)
        sc, sh = _bn_affine(b_ref[...], g_ref[...], be_ref[...], rm_ref[...],
                            rv_ref[...])
        h2 = jnp.maximum(z * sc + sh, 0.0)
        t = jnp.dot(h2, w3_ref[...], preferred_element_type=jnp.float32)
        tt_ref[...] = t * dinv

    p_spec = pl.BlockSpec((2, BR, F_IN), lambda i: (0, i, 0))
    h_spec = pl.BlockSpec((BR, F_IN), lambda i: (i, 0))
    v_spec = pl.BlockSpec((1, H), lambda i: (0, 0))
    return pl.pallas_call(
        body,
        grid=(N // BR,),
        in_specs=[
            p_spec, p_spec, h_spec, h_spec,
            pl.BlockSpec((BR, 1), lambda i: (i, 0)),
            pl.BlockSpec((H, H), lambda i: (0, 0)),
            v_spec, v_spec, v_spec, v_spec, v_spec,
            pl.BlockSpec((H, CP), lambda i: (0, 0)),
        ],
        out_specs=pl.BlockSpec((BR, CP), lambda i: (i, 0)),
        out_shape=jax.ShapeDtypeStruct((N, CP), jnp.float32),
    )(qa, qb, ha, hb, dinv, W2, b2, g2, be2, rm2, rv2, W3p)


def _tc_layer3(r, tt, dinv, b3p):
    def body(r_ref, tt_ref, dinv_ref, b_ref, out_ref):
        agg = (r_ref[0] + r_ref[1] + tt_ref[...]) * dinv_ref[...]
        logits = agg + b_ref[...]
        col = lax.broadcasted_iota(jnp.int32, (BR, CP), 1)
        masked = jnp.where(col < C, logits, -1e30)
        m = jnp.max(masked, axis=1, keepdims=True)
        lse = jnp.log(jnp.sum(jnp.exp(masked - m), axis=1, keepdims=True))
        out_ref[...] = logits - m - lse

    return pl.pallas_call(
        body,
        grid=(N // BR,),
        in_specs=[
            pl.BlockSpec((2, BR, CP), lambda i: (0, i, 0)),
            pl.BlockSpec((BR, CP), lambda i: (i, 0)),
            pl.BlockSpec((BR, 1), lambda i: (i, 0)),
            pl.BlockSpec((1, CP), lambda i: (0, 0)),
        ],
        out_specs=pl.BlockSpec((BR, CP), lambda i: (i, 0)),
        out_shape=jax.ShapeDtypeStruct((N, CP), jnp.float32),
    )(r, tt, dinv, b3p)


def kernel(x, W1, b1, g1, be1, rm1, rv1, W2, b2, g2, be2, rm2, rv2, W3, b3,
           edge_index):
    ei = edge_index.astype(jnp.int32)
    pad = jnp.arange(EPAD, dtype=jnp.int32)
    src = jnp.concatenate([ei[0], pad % N])
    dst = jnp.concatenate([ei[1], N + pad % NDEAD])
    dst3h = dst.reshape(NTILES, NBH, B)
    src3 = src.reshape(NTILES, NBH, B)
    dst3 = dst3h

    z16 = jnp.zeros((640, 16), jnp.float32)
    z128 = jnp.zeros((632, F_IN), jnp.float32)

    hist = _sc_hist(dst3h, z16).reshape(NTILES, NPAD, 1)
    dinv, xt = _tc_prep(hist, x)

    p = _sc_prop(xt, src3, dst3, z128, F_IN)
    ha, hb = _tc_layer1(p, xt, dinv, W1,
                        b1.reshape(1, H), g1.reshape(1, H),
                        be1.reshape(1, H), rm1.reshape(1, H),
                        rv1.reshape(1, H))

    qa = _sc_prop(ha, src3, dst3, z128, F_IN)
    qb = _sc_prop(hb, src3, dst3, z128, F_IN)
    W3p = jnp.pad(W3, ((0, 0), (0, CP - C)))
    tt = _tc_layer2(qa, qb, ha, hb, dinv, W2,
                    b2.reshape(1, H), g2.reshape(1, H), be2.reshape(1, H),
                    rm2.reshape(1, H), rv2.reshape(1, H), W3p)

    r = _sc_prop(tt, src3, dst3, z128, CP)
    b3p = jnp.pad(b3, (0, CP - C)).reshape(1, CP)
    out = _tc_layer3(r, tt, dinv, b3p)
    return out[:, :C]


# speedup vs baseline: 3.3872x; 1.4063x over previous
import functools

import jax
import jax.numpy as jnp
from jax import lax
from jax.experimental import pallas as pl
from jax.experimental.pallas import tpu as pltpu
from jax.experimental.pallas import tpu_sc as plsc

N = 10000
E = 320000
NPAD = 10240
NACC = 10112
B = 128
NBH = 80
NBI = NBH + 8
NTILES = 32
ETOT = NTILES * NBH * B
EPAD = ETOT - E
NDEAD = 112
BR = 1000
F_IN = 128
H = 256
C = 40
CP = 128

_MESH = plsc.VectorSubcoreMesh(core_axis_name="c", subcore_axis_name="s")
_SC_PARAMS = pltpu.CompilerParams(needs_layout_passes=False)


def _sc_hist(dst3, zrows):
    @functools.partial(
        pl.kernel,
        out_type=jax.ShapeDtypeStruct((NTILES, 640, 16), jnp.float32),
        mesh=_MESH,
        compiler_params=_SC_PARAMS,
        scratch_types=[
            pltpu.VMEM((NBH, B), jnp.int32),
            pltpu.VMEM((640, 16), jnp.float32),
        ],
    )
    def k(dst_hbm, z_hbm, out_hbm, dbuf, hist):
        c = lax.axis_index("c")
        s = lax.axis_index("s")
        t = c * 16 + s
        pltpu.sync_copy(dst_hbm.at[t], dbuf)
        pltpu.sync_copy(z_hbm, hist)

        @pl.loop(0, NBH)
        def _(j):
            @pl.loop(0, B, step=16)
            def _(kk):
                d = dbuf[j, pl.ds(kk, 16)]
                row = lax.shift_right_logical(d, 4)
                lane = lax.bitwise_and(d, 15)
                plsc.addupdate_scatter(
                    hist, [row, lane], jnp.ones((16,), jnp.float32))

        pltpu.sync_copy(hist, out_hbm.at[t])

    return k(dst3, zrows)


def _sc_prop(table, idx4, zrows, width):
    @functools.partial(
        pl.kernel,
        out_type=jax.ShapeDtypeStruct((2, NACC, width), jnp.float32),
        mesh=_MESH,
        scratch_types=[
            pltpu.VMEM((2, B), jnp.int32),
            pltpu.VMEM((2, B), jnp.int32),
            pltpu.VMEM((2, B), jnp.int32),
            pltpu.VMEM((2, B), jnp.int32),
            pltpu.VMEM((B, width), jnp.float32),
            pltpu.VMEM((B, width), jnp.float32),
            pltpu.VMEM_SHARED((NACC, width), jnp.float32),
            pltpu.SemaphoreType.DMA,
            pltpu.SemaphoreType.DMA,
            pltpu.SemaphoreType.DMA,
            pltpu.SemaphoreType.DMA,
            pltpu.SemaphoreType.DMA,
            pltpu.SemaphoreType.DMA,
        ],
    )
    def k(tab_hbm, idx_hbm, z_hbm, out_hbm,
          ib0, ib1, ib2, ib3, buf_a, buf_b, accum,
          is0, is1, is2, is3, sem_a, sem_b):
        c = lax.axis_index("c")
        s = lax.axis_index("s")
        t = c * 16 + s
        ti = idx_hbm.at[t]

        pltpu.async_copy(ti.at[0], ib0, is0)
        pltpu.async_copy(ti.at[1], ib1, is1)
        pltpu.async_copy(ti.at[2], ib2, is2)
        pltpu.async_copy(ti.at[3], ib3, is3)
        pltpu.sync_copy(z_hbm, accum.at[pl.ds(s * 632, 632)])
        plsc.subcore_barrier()
        pltpu.make_async_copy(ti.at[0], ib0, is0).wait()
        pltpu.async_copy(tab_hbm.at[ib0.at[0]], buf_a, sem_a)

        def gather(ib, isem, j, buf, sem):
            pltpu.make_async_copy(ti.at[j], ib, isem).wait()
            pltpu.async_copy(tab_hbm.at[ib.at[0]], buf, sem)

        def scatter(ib, buf, sem, jn, isem):
            pltpu.make_async_copy(tab_hbm.at[ib.at[0]], buf, sem).wait()
            pltpu.sync_copy(buf, accum.at[ib.at[1]], add=True)
            pltpu.async_copy(ti.at[jn], ib, isem)

        @pl.loop(0, NBH // 4)
        def _(kk):
            j = kk * 4
            gather(ib1, is1, j + 1, buf_b, sem_b)
            scatter(ib0, buf_a, sem_a, j + 4, is0)
            gather(ib2, is2, j + 2, buf_a, sem_a)
            scatter(ib1, buf_b, sem_b, j + 5, is1)
            gather(ib3, is3, j + 3, buf_b, sem_b)
            scatter(ib2, buf_a, sem_a, j + 6, is2)
            gather(ib0, is0, j + 4, buf_a, sem_a)
            scatter(ib3, buf_b, sem_b, j + 7, is3)

        pltpu.make_async_copy(tab_hbm.at[ib0.at[0]], buf_a, sem_a).wait()
        pltpu.make_async_copy(ti.at[NBH + 1], ib1, is1).wait()
        pltpu.make_async_copy(ti.at[NBH + 2], ib2, is2).wait()
        pltpu.make_async_copy(ti.at[NBH + 3], ib3, is3).wait()

        plsc.subcore_barrier()
        pltpu.sync_copy(accum.at[pl.ds(s * 632, 632)],
                        out_hbm.at[c, pl.ds(s * 632, 632)])

    return k(table, idx4, zrows)


def _tc_prep(hist, x):
    def body(h_ref, x_ref, dinv_ref, xt_ref):
        deg = jnp.sum(h_ref[...], axis=0) + 1.0
        dinv = lax.rsqrt(deg)
        dinv_ref[...] = dinv
        xt_ref[...] = x_ref[...] * dinv

    return pl.pallas_call(
        body,
        grid=(N // BR,),
        in_specs=[
            pl.BlockSpec((NTILES, BR, 1), lambda i: (0, i, 0)),
            pl.BlockSpec((BR, F_IN), lambda i: (i, 0)),
        ],
        out_specs=[
            pl.BlockSpec((BR, 1), lambda i: (i, 0)),
            pl.BlockSpec((BR, F_IN), lambda i: (i, 0)),
        ],
        out_shape=[
            jax.ShapeDtypeStruct((N, 1), jnp.float32),
            jax.ShapeDtypeStruct((N, F_IN), jnp.float32),
        ],
    )(hist, x)


def _bn_affine(b, g, be, rm, rv):
    sc = g * lax.rsqrt(rv + 1e-5)
    return sc, (b - rm) * sc + be


def _tc_layer1(parts, xt, dinv, W1, b1, g1, be1, rm1, rv1):
    def body(p_ref, xt_ref, dinv_ref, w_ref, b_ref, g_ref, be_ref, rm_ref,
             rv_ref, ha_ref, hb_ref):
        dinv = dinv_ref[...]
        agg = (p_ref[0] + p_ref[1] + xt_ref[...]) * dinv
        z = jnp.dot(agg, w_ref[...], preferred_element_type=jnp.float32)
        sc, sh = _bn_affine(b_ref[...], g_ref[...], be_ref[...], rm_ref[...],
                            rv_ref[...])
        h = jnp.maximum(z * sc + sh, 0.0) * dinv
        ha_ref[...] = h[:, :F_IN]
        hb_ref[...] = h[:, F_IN:]

    p_spec = pl.BlockSpec((2, BR, F_IN), lambda i: (0, i, 0))
    v_spec = pl.BlockSpec((1, H), lambda i: (0, 0))
    return pl.pallas_call(
        body,
        grid=(N // BR,),
        in_specs=[
            p_spec,
            pl.BlockSpec((BR, F_IN), lambda i: (i, 0)),
            pl.BlockSpec((BR, 1), lambda i: (i, 0)),
            pl.BlockSpec((F_IN, H), lambda i: (0, 0)),
            v_spec, v_spec, v_spec, v_spec, v_spec,
        ],
        out_specs=[
            pl.BlockSpec((BR, F_IN), lambda i: (i, 0)),
            pl.BlockSpec((BR, F_IN), lambda i: (i, 0)),
        ],
        out_shape=[
            jax.ShapeDtypeStruct((N, F_IN), jnp.float32),
            jax.ShapeDtypeStruct((N, F_IN), jnp.float32),
        ],
    )(parts, xt, dinv, W1, b1, g1, be1, rm1, rv1)


def _tc_layer2(qa, qb, ha, hb, dinv, W2, b2, g2, be2, rm2, rv2, W3p):
    def body(qa_ref, qb_ref, ha_ref, hb_ref, dinv_ref, w2_ref, b_ref, g_ref,
             be_ref, rm_ref, rv_ref, w3_ref, tt_ref):
        dinv = dinv_ref[...]
        agg_a = (qa_ref[0] + qa_ref[1] + ha_ref[...]) * dinv
        agg_b = (qb_ref[0] + qb_ref[1] + hb_ref[...]) * dinv
        agg = jnp.concatenate([agg_a, agg_b], axis=1)
        z = jnp.dot(agg, w2_ref[...], preferred_element_type=jnp.float32)
        sc, sh = _bn_affine(b_ref[...], g_ref[...], be_ref[...], rm_ref[...],
                            rv_ref[...])
        h2 = jnp.maximum(z * sc + sh, 0.0)
        t = jnp.dot(h2, w3_ref[...], preferred_element_type=jnp.float32)
        tt_ref[...] = t * dinv

    p_spec = pl.BlockSpec((2, BR, F_IN), lambda i: (0, i, 0))
    h_spec = pl.BlockSpec((BR, F_IN), lambda i: (i, 0))
    v_spec = pl.BlockSpec((1, H), lambda i: (0, 0))
    return pl.pallas_call(
        body,
        grid=(N // BR,),
        in_specs=[
            p_spec, p_spec, h_spec, h_spec,
            pl.BlockSpec((BR, 1), lambda i: (i, 0)),
            pl.BlockSpec((H, H), lambda i: (0, 0)),
            v_spec, v_spec, v_spec, v_spec, v_spec,
            pl.BlockSpec((H, CP), lambda i: (0, 0)),
        ],
        out_specs=pl.BlockSpec((BR, CP), lambda i: (i, 0)),
        out_shape=jax.ShapeDtypeStruct((N, CP), jnp.float32),
    )(qa, qb, ha, hb, dinv, W2, b2, g2, be2, rm2, rv2, W3p)


def _tc_layer3(r, tt, dinv, b3p):
    def body(r_ref, tt_ref, dinv_ref, b_ref, out_ref):
        agg = (r_ref[0] + r_ref[1] + tt_ref[...]) * dinv_ref[...]
        logits = agg + b_ref[...]
        col = lax.broadcasted_iota(jnp.int32, (BR, CP), 1)
        masked = jnp.where(col < C, logits, -1e30)
        m = jnp.max(masked, axis=1, keepdims=True)
        lse = jnp.log(jnp.sum(jnp.exp(masked - m), axis=1, keepdims=True))
        out_ref[...] = logits - m - lse

    return pl.pallas_call(
        body,
        grid=(N // BR,),
        in_specs=[
            pl.BlockSpec((2, BR, CP), lambda i: (0, i, 0)),
            pl.BlockSpec((BR, CP), lambda i: (i, 0)),
            pl.BlockSpec((BR, 1), lambda i: (i, 0)),
            pl.BlockSpec((1, CP), lambda i: (0, 0)),
        ],
        out_specs=pl.BlockSpec((BR, CP), lambda i: (i, 0)),
        out_shape=jax.ShapeDtypeStruct((N, CP), jnp.float32),
    )(r, tt, dinv, b3p)


def kernel(x, W1, b1, g1, be1, rm1, rv1, W2, b2, g2, be2, rm2, rv2, W3, b3,
           edge_index):
    ei = edge_index.astype(jnp.int32)
    pad = jnp.arange(EPAD, dtype=jnp.int32)
    src = jnp.concatenate([ei[0], pad % N])
    dst = jnp.concatenate([ei[1], N + pad % NDEAD])
    dst3h = dst.reshape(NTILES, NBH, B)
    idx_main = jnp.stack(
        [src.reshape(NTILES, NBH, B), dst.reshape(NTILES, NBH, B)], axis=2)
    spad = jnp.arange(NTILES * (NBI - NBH) * B, dtype=jnp.int32)
    slack = jnp.stack(
        [(spad % N).reshape(NTILES, NBI - NBH, B),
         (N + spad % NDEAD).reshape(NTILES, NBI - NBH, B)], axis=2)
    idx4 = jnp.concatenate([idx_main, slack], axis=1)

    z16 = jnp.zeros((640, 16), jnp.float32)
    z128 = jnp.zeros((632, F_IN), jnp.float32)

    hist = _sc_hist(dst3h, z16).reshape(NTILES, NPAD, 1)
    dinv, xt = _tc_prep(hist, x)

    p = _sc_prop(xt, idx4, z128, F_IN)
    ha, hb = _tc_layer1(p, xt, dinv, W1,
                        b1.reshape(1, H), g1.reshape(1, H),
                        be1.reshape(1, H), rm1.reshape(1, H),
                        rv1.reshape(1, H))

    qa = _sc_prop(ha, idx4, z128, F_IN)
    qb = _sc_prop(hb, idx4, z128, F_IN)
    W3p = jnp.pad(W3, ((0, 0), (0, CP - C)))
    tt = _tc_layer2(qa, qb, ha, hb, dinv, W2,
                    b2.reshape(1, H), g2.reshape(1, H), be2.reshape(1, H),
                    rm2.reshape(1, H), rv2.reshape(1, H), W3p)

    r = _sc_prop(tt, idx4, z128, CP)
    b3p = jnp.pad(b3, (0, CP - C)).reshape(1, CP)
    out = _tc_layer3(r, tt, dinv, b3p)
    return out[:, :C]


# speedup vs baseline: 4.0960x; 1.2093x over previous
import functools

import jax
import jax.numpy as jnp
from jax import lax
from jax.experimental import pallas as pl
from jax.experimental.pallas import tpu as pltpu
from jax.experimental.pallas import tpu_sc as plsc

N = 10000
E = 320000
NPAD = 10240
NACC = 10112
B = 128
NBH = 80
NBI = NBH + 8
NTILES = 32
ETOT = NTILES * NBH * B
EPAD = ETOT - E
NDEAD = 112
BR = 1000
F_IN = 128
H = 256
C = 40
CP = 128

_MESH = plsc.VectorSubcoreMesh(core_axis_name="c", subcore_axis_name="s")
_SC_PARAMS = pltpu.CompilerParams(needs_layout_passes=False)


def _sc_hist(dst3, zrows):
    @functools.partial(
        pl.kernel,
        out_type=jax.ShapeDtypeStruct((NTILES, 640, 16), jnp.float32),
        mesh=_MESH,
        compiler_params=_SC_PARAMS,
        scratch_types=[
            pltpu.VMEM((NBH, B), jnp.int32),
            pltpu.VMEM((640, 16), jnp.float32),
        ],
    )
    def k(dst_hbm, z_hbm, out_hbm, dbuf, hist):
        c = lax.axis_index("c")
        s = lax.axis_index("s")
        t = c * 16 + s
        pltpu.sync_copy(dst_hbm.at[t], dbuf)
        pltpu.sync_copy(z_hbm, hist)

        @pl.loop(0, NBH)
        def _(j):
            @pl.loop(0, B, step=16)
            def _(kk):
                d = dbuf[j, pl.ds(kk, 16)]
                row = lax.shift_right_logical(d, 4)
                lane = lax.bitwise_and(d, 15)
                plsc.addupdate_scatter(
                    hist, [row, lane], jnp.ones((16,), jnp.float32))

        pltpu.sync_copy(hist, out_hbm.at[t])

    return k(dst3, zrows)


def _sc_prop(table, idx4, zrows, width):
    @functools.partial(
        pl.kernel,
        out_type=jax.ShapeDtypeStruct((2, NACC, width), jnp.float32),
        mesh=_MESH,
        scratch_types=[
            pltpu.VMEM((2, B), jnp.int32),
            pltpu.VMEM((2, B), jnp.int32),
            pltpu.VMEM((2, B), jnp.int32),
            pltpu.VMEM((2, B), jnp.int32),
            pltpu.VMEM((B, width), jnp.float32),
            pltpu.VMEM((B, width), jnp.float32),
            pltpu.VMEM_SHARED((NACC, width), jnp.float32),
            pltpu.SemaphoreType.DMA,
            pltpu.SemaphoreType.DMA,
            pltpu.SemaphoreType.DMA,
            pltpu.SemaphoreType.DMA,
            pltpu.SemaphoreType.DMA,
            pltpu.SemaphoreType.DMA,
        ],
    )
    def k(tab_hbm, idx_hbm, z_hbm, out_hbm,
          ib0, ib1, ib2, ib3, buf_a, buf_b, accum,
          is0, is1, is2, is3, sem_a, sem_b):
        c = lax.axis_index("c")
        s = lax.axis_index("s")
        t = c * 16 + s
        ti = idx_hbm.at[t]

        pltpu.async_copy(ti.at[0], ib0, is0)
        pltpu.async_copy(ti.at[1], ib1, is1)
        pltpu.async_copy(ti.at[2], ib2, is2)
        pltpu.async_copy(ti.at[3], ib3, is3)
        pltpu.sync_copy(z_hbm, accum.at[pl.ds(s * 632, 632)])
        plsc.subcore_barrier()
        pltpu.make_async_copy(ti.at[0], ib0, is0).wait()
        pltpu.async_copy(tab_hbm.at[ib0.at[0]], buf_a, sem_a)

        def gather(ib, isem, j, buf, sem):
            pltpu.make_async_copy(ti.at[j], ib, isem).wait()
            pltpu.async_copy(tab_hbm.at[ib.at[0]], buf, sem)

        def scatter(ib, buf, sem, jn, isem):
            pltpu.make_async_copy(tab_hbm.at[ib.at[0]], buf, sem).wait()
            pltpu.sync_copy(buf, accum.at[ib.at[1]], add=True)
            pltpu.async_copy(ti.at[jn], ib, isem)

        @pl.loop(0, NBH // 4)
        def _(kk):
            j = kk * 4
            gather(ib1, is1, j + 1, buf_b, sem_b)
            scatter(ib0, buf_a, sem_a, j + 4, is0)
            gather(ib2, is2, j + 2, buf_a, sem_a)
            scatter(ib1, buf_b, sem_b, j + 5, is1)
            gather(ib3, is3, j + 3, buf_b, sem_b)
            scatter(ib2, buf_a, sem_a, j + 6, is2)
            gather(ib0, is0, j + 4, buf_a, sem_a)
            scatter(ib3, buf_b, sem_b, j + 7, is3)

        pltpu.make_async_copy(tab_hbm.at[ib0.at[0]], buf_a, sem_a).wait()
        pltpu.make_async_copy(ti.at[NBH + 1], ib1, is1).wait()
        pltpu.make_async_copy(ti.at[NBH + 2], ib2, is2).wait()
        pltpu.make_async_copy(ti.at[NBH + 3], ib3, is3).wait()

        plsc.subcore_barrier()
        pltpu.sync_copy(accum.at[pl.ds(s * 632, 632)],
                        out_hbm.at[c, pl.ds(s * 632, 632)])

    return k(table, idx4, zrows)


def _tc_hsum(hist):
    def body(h_ref, o_ref):
        o_ref[...] = jnp.sum(h_ref[...], axis=0)

    return pl.pallas_call(
        body,
        out_shape=jax.ShapeDtypeStruct((640, 16), jnp.float32),
    )(hist)


def _tc_prep(hist, x):
    def body(h_ref, x_ref, dinv_ref, xt_ref):
        deg = h_ref[...] + 1.0
        dinv = lax.rsqrt(deg)
        dinv_ref[...] = dinv
        xt_ref[...] = x_ref[...] * dinv

    return pl.pallas_call(
        body,
        grid=(N // BR,),
        in_specs=[
            pl.BlockSpec((BR, 1), lambda i: (i, 0)),
            pl.BlockSpec((BR, F_IN), lambda i: (i, 0)),
        ],
        out_specs=[
            pl.BlockSpec((BR, 1), lambda i: (i, 0)),
            pl.BlockSpec((BR, F_IN), lambda i: (i, 0)),
        ],
        out_shape=[
            jax.ShapeDtypeStruct((N, 1), jnp.float32),
            jax.ShapeDtypeStruct((N, F_IN), jnp.float32),
        ],
    )(hist, x)


def _bn_affine(b, g, be, rm, rv):
    sc = g * lax.rsqrt(rv + 1e-5)
    return sc, (b - rm) * sc + be


def _tc_layer1(parts, xt, dinv, W1, b1, g1, be1, rm1, rv1):
    def body(p_ref, xt_ref, dinv_ref, w_ref, b_ref, g_ref, be_ref, rm_ref,
             rv_ref, ha_ref, hb_ref):
        dinv = dinv_ref[...]
        agg = (p_ref[0] + p_ref[1] + xt_ref[...]) * dinv
        z = jnp.dot(agg, w_ref[...], preferred_element_type=jnp.float32)
        sc, sh = _bn_affine(b_ref[...], g_ref[...], be_ref[...], rm_ref[...],
                            rv_ref[...])
        h = jnp.maximum(z * sc + sh, 0.0) * dinv
        ha_ref[...] = h[:, :F_IN]
        hb_ref[...] = h[:, F_IN:]

    p_spec = pl.BlockSpec((2, BR, F_IN), lambda i: (0, i, 0))
    v_spec = pl.BlockSpec((1, H), lambda i: (0, 0))
    return pl.pallas_call(
        body,
        grid=(N // BR,),
        in_specs=[
            p_spec,
            pl.BlockSpec((BR, F_IN), lambda i: (i, 0)),
            pl.BlockSpec((BR, 1), lambda i: (i, 0)),
            pl.BlockSpec((F_IN, H), lambda i: (0, 0)),
            v_spec, v_spec, v_spec, v_spec, v_spec,
        ],
        out_specs=[
            pl.BlockSpec((BR, F_IN), lambda i: (i, 0)),
            pl.BlockSpec((BR, F_IN), lambda i: (i, 0)),
        ],
        out_shape=[
            jax.ShapeDtypeStruct((N, F_IN), jnp.float32),
            jax.ShapeDtypeStruct((N, F_IN), jnp.float32),
        ],
    )(parts, xt, dinv, W1, b1, g1, be1, rm1, rv1)


def _tc_layer2(qa, qb, ha, hb, dinv, W2, b2, g2, be2, rm2, rv2, W3p):
    def body(qa_ref, qb_ref, ha_ref, hb_ref, dinv_ref, w2_ref, b_ref, g_ref,
             be_ref, rm_ref, rv_ref, w3_ref, tt_ref):
        dinv = dinv_ref[...]
        agg_a = (qa_ref[0] + qa_ref[1] + ha_ref[...]) * dinv
        agg_b = (qb_ref[0] + qb_ref[1] + hb_ref[...]) * dinv
        agg = jnp.concatenate([agg_a, agg_b], axis=1)
        z = jnp.dot(agg, w2_ref[...], preferred_element_type=jnp.float32)
        sc, sh = _bn_affine(b_ref[...], g_ref[...], be_ref[...], rm_ref[...],
                            rv_ref[...])
        h2 = jnp.maximum(z * sc + sh, 0.0)
        t = jnp.dot(h2, w3_ref[...], preferred_element_type=jnp.float32)
        tt_ref[...] = t * dinv

    p_spec = pl.BlockSpec((2, BR, F_IN), lambda i: (0, i, 0))
    h_spec = pl.BlockSpec((BR, F_IN), lambda i: (i, 0))
    v_spec = pl.BlockSpec((1, H), lambda i: (0, 0))
    return pl.pallas_call(
        body,
        grid=(N // BR,),
        in_specs=[
            p_spec, p_spec, h_spec, h_spec,
            pl.BlockSpec((BR, 1), lambda i: (i, 0)),
            pl.BlockSpec((H, H), lambda i: (0, 0)),
            v_spec, v_spec, v_spec, v_spec, v_spec,
            pl.BlockSpec((H, CP), lambda i: (0, 0)),
        ],
        out_specs=pl.BlockSpec((BR, CP), lambda i: (i, 0)),
        out_shape=jax.ShapeDtypeStruct((N, CP), jnp.float32),
    )(qa, qb, ha, hb, dinv, W2, b2, g2, be2, rm2, rv2, W3p)


def _tc_layer3(r, tt, dinv, b3p):
    def body(r_ref, tt_ref, dinv_ref, b_ref, out_ref):
        agg = (r_ref[0] + r_ref[1] + tt_ref[...]) * dinv_ref[...]
        logits = agg + b_ref[...]
        col = lax.broadcasted_iota(jnp.int32, (BR, CP), 1)
        masked = jnp.where(col < C, logits, -1e30)
        m = jnp.max(masked, axis=1, keepdims=True)
        lse = jnp.log(jnp.sum(jnp.exp(masked - m), axis=1, keepdims=True))
        out_ref[...] = logits - m - lse

    return pl.pallas_call(
        body,
        grid=(N // BR,),
        in_specs=[
            pl.BlockSpec((2, BR, CP), lambda i: (0, i, 0)),
            pl.BlockSpec((BR, CP), lambda i: (i, 0)),
            pl.BlockSpec((BR, 1), lambda i: (i, 0)),
            pl.BlockSpec((1, CP), lambda i: (0, 0)),
        ],
        out_specs=pl.BlockSpec((BR, CP), lambda i: (i, 0)),
        out_shape=jax.ShapeDtypeStruct((N, CP), jnp.float32),
    )(r, tt, dinv, b3p)


def kernel(x, W1, b1, g1, be1, rm1, rv1, W2, b2, g2, be2, rm2, rv2, W3, b3,
           edge_index):
    ei = edge_index.astype(jnp.int32)
    pad = jnp.arange(EPAD, dtype=jnp.int32)
    src = jnp.concatenate([ei[0], pad % N])
    dst = jnp.concatenate([ei[1], N + pad % NDEAD])
    dst3h = dst.reshape(NTILES, NBH, B)
    idx_main = jnp.stack(
        [src.reshape(NTILES, NBH, B), dst.reshape(NTILES, NBH, B)], axis=2)
    spad = jnp.arange(NTILES * (NBI - NBH) * B, dtype=jnp.int32)
    slack = jnp.stack(
        [(spad % N).reshape(NTILES, NBI - NBH, B),
         (N + spad % NDEAD).reshape(NTILES, NBI - NBH, B)], axis=2)
    idx4 = jnp.concatenate([idx_main, slack], axis=1)

    z16 = jnp.zeros((640, 16), jnp.float32)
    z128 = jnp.zeros((632, F_IN), jnp.float32)

    hist = _tc_hsum(_sc_hist(dst3h, z16)).reshape(NPAD, 1)
    dinv, xt = _tc_prep(hist, x)

    p = _sc_prop(xt, idx4, z128, F_IN)
    ha, hb = _tc_layer1(p, xt, dinv, W1,
                        b1.reshape(1, H), g1.reshape(1, H),
                        be1.reshape(1, H), rm1.reshape(1, H),
                        rv1.reshape(1, H))

    qa = _sc_prop(ha, idx4, z128, F_IN)
    qb = _sc_prop(hb, idx4, z128, F_IN)
    W3p = jnp.pad(W3, ((0, 0), (0, CP - C)))
    tt = _tc_layer2(qa, qb, ha, hb, dinv, W2,
                    b2.reshape(1, H), g2.reshape(1, H), be2.reshape(1, H),
                    rm2.reshape(1, H), rv2.reshape(1, H), W3p)

    r = _sc_prop(tt, idx4, z128, CP)
    b3p = jnp.pad(b3, (0, CP - C)).reshape(1, CP)
    out = _tc_layer3(r, tt, dinv, b3p)
    return out[:, :C]
